# baseline (speedup 1.0000x reference)
import functools

import jax
import jax.numpy as jnp
from jax import lax
from jax.experimental import pallas as pl
from jax.experimental.pallas import tpu as pltpu
from jax.experimental.pallas import tpu_sc as plsc

N = 10000
E = 320000
D = 128

NC = 2
NS = 16
NW = NC * NS

CHUNK = 128
EPW = E // NW
NCHUNK = -(-EPW // CHUNK)
EPW_PAD = NCHUNK * CHUNK
NCH0 = 48
NCH1 = 112
NCHMAX = max(NCH0, NCH1)
TOTCH = NS * (NCH0 + NCH1)
RPT = 8 * -(-(N + 1) // (8 * NS))
N_PAD = RPT * NS

BM = 1000
NB = N // BM

@functools.cache
def _mesh():
    return plsc.VectorSubcoreMesh(core_axis_name="c", subcore_axis_name="s",
                                  num_cores=NC, num_subcores=NS)


def _sc_degree(dstp, ones, zeros):
    @functools.partial(
        pl.kernel,
        out_type=jax.ShapeDtypeStruct((NC, N_PAD, D), jnp.float32),
        mesh=_mesh(),
        scratch_types=[
            pltpu.VMEM((NCHUNK, CHUNK), jnp.int32),
            pltpu.VMEM((CHUNK, D), jnp.float32),
            pltpu.VMEM_SHARED((N_PAD, D), jnp.float32),
        ],
    )
    def k(dst_hbm, ones_hbm, z_hbm, out_hbm, dst_v, ones_v, acc_sh):
        c = lax.axis_index("c")
        s = lax.axis_index("s")
        wid = s * NC + c
        pltpu.sync_copy(dst_hbm.at[wid], dst_v)
        pltpu.sync_copy(ones_hbm, ones_v)
        rows = pl.ds(s * RPT, RPT)
        pltpu.sync_copy(z_hbm.at[rows], acc_sh.at[rows])
        plsc.subcore_barrier()

        @pl.loop(0, NCHUNK)
        def _(j):
            pltpu.sync_copy(ones_v, acc_sh.at[dst_v.at[j]], add=True)

        plsc.subcore_barrier()
        pltpu.sync_copy(acc_sh.at[rows], out_hbm.at[c, rows])

    return k(dstp, ones, zeros)


def _sc_aggregate(g, srcp, dstp, zeros):
    @functools.partial(
        pl.kernel,
        out_type=jax.ShapeDtypeStruct((NC, N_PAD, D), jnp.float32),
        mesh=_mesh(),
        scratch_types=[
            pltpu.VMEM((NCHMAX, CHUNK), jnp.int32),
            pltpu.VMEM((NCHMAX, CHUNK), jnp.int32),
            pltpu.VMEM((CHUNK, D), jnp.float32),
            pltpu.VMEM_SHARED((N_PAD, D), jnp.float32),
        ],
    )
    def k(g_hbm, src_hbm, dst_hbm, z_hbm, out_hbm, src_v, dst_v, rows_v, acc_sh):
        c = lax.axis_index("c")
        s = lax.axis_index("s")
        rows = pl.ds(s * RPT, RPT)
        pltpu.sync_copy(z_hbm.at[rows], acc_sh.at[rows])

        start = jnp.where(c == 0, s * NCH0, NS * NCH0 + s * NCH1)
        nch = jnp.where(c == 0, NCH0, NCH1)
        pltpu.sync_copy(src_hbm.at[pl.ds(start, NCHMAX)], src_v)
        pltpu.sync_copy(dst_hbm.at[pl.ds(start, NCHMAX)], dst_v)
        plsc.subcore_barrier()

        @pl.loop(0, nch)
        def _(j):
            pltpu.sync_copy(g_hbm.at[src_v.at[j]], rows_v)
            pltpu.sync_copy(rows_v, acc_sh.at[dst_v.at[j]], add=True)

        plsc.subcore_barrier()
        pltpu.sync_copy(acc_sh.at[rows], out_hbm.at[c, rows])

    return k(g, srcp, dstp, zeros)


def _mm_body(x_ref, w_ref, o_ref):
    o_ref[...] = jnp.dot(x_ref[...], w_ref[...], preferred_element_type=jnp.float32)


def _mm(x, w):
    return pl.pallas_call(
        _mm_body,
        grid=(NB,),
        in_specs=[
            pl.BlockSpec((BM, D), lambda i: (i, 0)),
            pl.BlockSpec((D, D), lambda i: (0, 0)),
        ],
        out_specs=pl.BlockSpec((BM, D), lambda i: (i, 0)),
        out_shape=jax.ShapeDtypeStruct((N, D), jnp.float32),
    )(x, w)


def _dis_block(degp_ref):
    deg = 1.0 + degp_ref[0, :, 0:1]
    for i in range(1, NC):
        deg = deg + degp_ref[i, :, 0:1]
    return lax.rsqrt(deg)


def _scale_body(h_ref, degp_ref, o_ref):
    o_ref[...] = h_ref[...] * _dis_block(degp_ref)


def _scale(h, degp):
    return pl.pallas_call(
        _scale_body,
        grid=(NB,),
        in_specs=[
            pl.BlockSpec((BM, D), lambda i: (i, 0)),
            pl.BlockSpec((NC, BM, D), lambda i: (0, i, 0)),
        ],
        out_specs=pl.BlockSpec((BM, D), lambda i: (i, 0)),
        out_shape=jax.ShapeDtypeStruct((N, D), jnp.float32),
    )(h, degp)


def _out1_block(p_ref, g1_ref, degp_ref, b1_ref):
    agg = p_ref[0] + g1_ref[...]
    for i in range(1, NC):
        agg = agg + p_ref[i]
    return agg * _dis_block(degp_ref) + b1_ref[...]


def _stats_body(p_ref, g1_ref, degp_ref, b1_ref, o_ref):
    i = pl.program_id(0)
    out1 = _out1_block(p_ref, g1_ref, degp_ref, b1_ref)

    @pl.when(i == 0)
    def _():
        o_ref[...] = jnp.zeros_like(o_ref)

    o_ref[0:1, :] += jnp.sum(out1, axis=0, keepdims=True)
    o_ref[1:2, :] += jnp.sum(out1 * out1, axis=0, keepdims=True)


def _stats(p, g1, degp, b1):
    return pl.pallas_call(
        _stats_body,
        grid=(NB,),
        in_specs=[
            pl.BlockSpec((NC, BM, D), lambda i: (0, i, 0)),
            pl.BlockSpec((BM, D), lambda i: (i, 0)),
            pl.BlockSpec((NC, BM, D), lambda i: (0, i, 0)),
            pl.BlockSpec((1, D), lambda i: (0, 0)),
        ],
        out_specs=pl.BlockSpec((8, D), lambda i: (0, 0)),
        out_shape=jax.ShapeDtypeStruct((8, D), jnp.float32),
    )(p, g1, degp, b1)


def _apply_body(p_ref, g1_ref, degp_ref, b1_ref, gm_ref, bt_ref, w2_ref, st_ref, o_ref):
    out1 = _out1_block(p_ref, g1_ref, degp_ref, b1_ref)
    mean = st_ref[0:1, :] * (1.0 / N)
    var = st_ref[1:2, :] * (1.0 / N) - mean * mean
    h = (out1 - mean) * lax.rsqrt(var + 1e-5) * gm_ref[...] + bt_ref[...]
    h = jnp.maximum(h, 0.0)
    o_ref[...] = jnp.dot(h, w2_ref[...], preferred_element_type=jnp.float32) * _dis_block(degp_ref)


def _apply(p, g1, degp, b1, gamma, beta, w2, st):
    return pl.pallas_call(
        _apply_body,
        grid=(NB,),
        in_specs=[
            pl.BlockSpec((NC, BM, D), lambda i: (0, i, 0)),
            pl.BlockSpec((BM, D), lambda i: (i, 0)),
            pl.BlockSpec((NC, BM, D), lambda i: (0, i, 0)),
            pl.BlockSpec((1, D), lambda i: (0, 0)),
            pl.BlockSpec((1, D), lambda i: (0, 0)),
            pl.BlockSpec((1, D), lambda i: (0, 0)),
            pl.BlockSpec((D, D), lambda i: (0, 0)),
            pl.BlockSpec((8, D), lambda i: (0, 0)),
        ],
        out_specs=pl.BlockSpec((BM, D), lambda i: (i, 0)),
        out_shape=jax.ShapeDtypeStruct((N, D), jnp.float32),
    )(p, g1, degp, b1, gamma, beta, w2, st)


def _final_body(q_ref, g2_ref, degp_ref, b2_ref, wt_ref, lb_ref, o_ref):
    out2 = _out1_block(q_ref, g2_ref, degp_ref, b2_ref)
    z = jnp.dot(out2, wt_ref[...], preferred_element_type=jnp.float32) + lb_ref[...]
    m = jnp.max(z, axis=1, keepdims=True)
    zs = z - m
    lse = jnp.log(jnp.sum(jnp.exp(zs), axis=1, keepdims=True))
    o_ref[...] = zs - lse


def _final(q, g2, degp, b2, linWT, linb):
    return pl.pallas_call(
        _final_body,
        grid=(NB,),
        in_specs=[
            pl.BlockSpec((NC, BM, D), lambda i: (0, i, 0)),
            pl.BlockSpec((BM, D), lambda i: (i, 0)),
            pl.BlockSpec((NC, BM, D), lambda i: (0, i, 0)),
            pl.BlockSpec((1, D), lambda i: (0, 0)),
            pl.BlockSpec((D, D), lambda i: (0, 0)),
            pl.BlockSpec((1, D), lambda i: (0, 0)),
        ],
        out_specs=pl.BlockSpec((BM, D), lambda i: (i, 0)),
        out_shape=jax.ShapeDtypeStruct((N, D), jnp.float32),
    )(q, g2, degp, b2, linWT, linb)


def kernel(x, edge_index, W1, b1, gamma, beta, W2, b2, linW, linb):
    src = edge_index[0]
    dst = edge_index[1]
    pad = NW * EPW_PAD - E
    srcp = jnp.concatenate([src, jnp.zeros((pad,), src.dtype)]).reshape(NW, NCHUNK, CHUNK)
    dstp = jnp.concatenate([dst, jnp.full((pad,), N, dst.dtype)]).reshape(NW, NCHUNK, CHUNK)
    padq = TOTCH * CHUNK - E
    srcq = jnp.concatenate([src, jnp.zeros((padq,), src.dtype)]).reshape(TOTCH, CHUNK)
    dstq = jnp.concatenate([dst, jnp.full((padq,), N, dst.dtype)]).reshape(TOTCH, CHUNK)

    zeros = jnp.zeros((N_PAD, D), jnp.float32)
    ones = jnp.ones((CHUNK, D), jnp.float32)

    degp = _sc_degree(dstp, ones, zeros)
    h1 = _mm(x, W1)
    g1 = _scale(h1, degp)
    p = _sc_aggregate(g1, srcq, dstq, zeros)
    b1r = b1.reshape(1, D)
    st = _stats(p, g1, degp, b1r)
    g2 = _apply(p, g1, degp, b1r, gamma.reshape(1, D), beta.reshape(1, D), W2, st)
    q = _sc_aggregate(g2, srcq, dstq, zeros)
    return _final(q, g2, degp, b2.reshape(1, D), linW.T, linb.reshape(1, D))


# speedup vs baseline: 1.2398x; 1.2398x over previous
import functools

import jax
import jax.numpy as jnp
from jax import lax
from jax.experimental import pallas as pl
from jax.experimental.pallas import tpu as pltpu
from jax.experimental.pallas import tpu_sc as plsc

N = 10000
E = 320000
D = 128

NC = 2
NS = 16
NW = NC * NS

CHUNK = 128
EPW = E // NW
NCHUNK = -(-EPW // CHUNK)
EPW_PAD = NCHUNK * CHUNK
NCH0 = 112
NCH1 = 48
NCHMAX = max(NCH0, NCH1)
TOTCH = NS * (NCH0 + NCH1)
RPT = 8 * -(-(N + 1) // (8 * NS))
N_PAD = RPT * NS

BM = 1000
NB = N // BM

@functools.cache
def _mesh():
    return plsc.VectorSubcoreMesh(core_axis_name="c", subcore_axis_name="s",
                                  num_cores=NC, num_subcores=NS)


def _sc_degree(dstp, ones, zeros):
    @functools.partial(
        pl.kernel,
        out_type=jax.ShapeDtypeStruct((NC, N_PAD, D), jnp.float32),
        mesh=_mesh(),
        scratch_types=[
            pltpu.VMEM((NCHUNK, CHUNK), jnp.int32),
            pltpu.VMEM((CHUNK, D), jnp.float32),
            pltpu.VMEM_SHARED((N_PAD, D), jnp.float32),
        ],
    )
    def k(dst_hbm, ones_hbm, z_hbm, out_hbm, dst_v, ones_v, acc_sh):
        c = lax.axis_index("c")
        s = lax.axis_index("s")
        wid = s * NC + c
        pltpu.sync_copy(dst_hbm.at[wid], dst_v)
        pltpu.sync_copy(ones_hbm, ones_v)
        rows = pl.ds(s * RPT, RPT)
        pltpu.sync_copy(z_hbm.at[rows], acc_sh.at[rows])
        plsc.subcore_barrier()

        @pl.loop(0, NCHUNK)
        def _(j):
            pltpu.sync_copy(ones_v, acc_sh.at[dst_v.at[j]], add=True)

        plsc.subcore_barrier()
        pltpu.sync_copy(acc_sh.at[rows], out_hbm.at[c, rows])

    return k(dstp, ones, zeros)


def _sc_aggregate(g, srcp, dstp, zeros):
    @functools.partial(
        pl.kernel,
        out_type=jax.ShapeDtypeStruct((NC, N_PAD, D), jnp.float32),
        mesh=_mesh(),
        scratch_types=[
            pltpu.VMEM((NCHMAX, CHUNK), jnp.int32),
            pltpu.VMEM((NCHMAX, CHUNK), jnp.int32),
            pltpu.VMEM((CHUNK, D), jnp.float32),
            pltpu.VMEM_SHARED((N_PAD, D), jnp.float32),
        ],
    )
    def k(g_hbm, src_hbm, dst_hbm, z_hbm, out_hbm, src_v, dst_v, rows_v, acc_sh):
        c = lax.axis_index("c")
        s = lax.axis_index("s")
        rows = pl.ds(s * RPT, RPT)
        pltpu.sync_copy(z_hbm.at[rows], acc_sh.at[rows])

        def run_core(start, nch):
            pltpu.sync_copy(src_hbm.at[pl.ds(start, nch)],
                            src_v.at[pl.ds(0, nch)])
            pltpu.sync_copy(dst_hbm.at[pl.ds(start, nch)],
                            dst_v.at[pl.ds(0, nch)])

            @pl.loop(0, nch)
            def _(j):
                pltpu.sync_copy(g_hbm.at[src_v.at[j]], rows_v)
                pltpu.sync_copy(rows_v, acc_sh.at[dst_v.at[j]], add=True)

        @pl.when(c == 0)
        def _():
            run_core(s * NCH0, NCH0)

        @pl.when(c == 1)
        def _():
            run_core(NS * NCH0 + s * NCH1, NCH1)

        plsc.subcore_barrier()
        pltpu.sync_copy(acc_sh.at[rows], out_hbm.at[c, rows])

    return k(g, srcp, dstp, zeros)


def _mm_body(x_ref, w_ref, o_ref):
    o_ref[...] = jnp.dot(x_ref[...], w_ref[...], preferred_element_type=jnp.float32)


def _mm(x, w):
    return pl.pallas_call(
        _mm_body,
        grid=(NB,),
        in_specs=[
            pl.BlockSpec((BM, D), lambda i: (i, 0)),
            pl.BlockSpec((D, D), lambda i: (0, 0)),
        ],
        out_specs=pl.BlockSpec((BM, D), lambda i: (i, 0)),
        out_shape=jax.ShapeDtypeStruct((N, D), jnp.float32),
    )(x, w)


def _dis_block(degp_ref):
    deg = 1.0 + degp_ref[0, :, 0:1]
    for i in range(1, NC):
        deg = deg + degp_ref[i, :, 0:1]
    return lax.rsqrt(deg)


def _scale_body(h_ref, degp_ref, o_ref):
    o_ref[...] = h_ref[...] * _dis_block(degp_ref)


def _scale(h, degp):
    return pl.pallas_call(
        _scale_body,
        grid=(NB,),
        in_specs=[
            pl.BlockSpec((BM, D), lambda i: (i, 0)),
            pl.BlockSpec((NC, BM, D), lambda i: (0, i, 0)),
        ],
        out_specs=pl.BlockSpec((BM, D), lambda i: (i, 0)),
        out_shape=jax.ShapeDtypeStruct((N, D), jnp.float32),
    )(h, degp)


def _out1_block(p_ref, g1_ref, degp_ref, b1_ref):
    agg = p_ref[0] + g1_ref[...]
    for i in range(1, NC):
        agg = agg + p_ref[i]
    return agg * _dis_block(degp_ref) + b1_ref[...]


def _stats_body(p_ref, g1_ref, degp_ref, b1_ref, o_ref):
    i = pl.program_id(0)
    out1 = _out1_block(p_ref, g1_ref, degp_ref, b1_ref)

    @pl.when(i == 0)
    def _():
        o_ref[...] = jnp.zeros_like(o_ref)

    o_ref[0:1, :] += jnp.sum(out1, axis=0, keepdims=True)
    o_ref[1:2, :] += jnp.sum(out1 * out1, axis=0, keepdims=True)


def _stats(p, g1, degp, b1):
    return pl.pallas_call(
        _stats_body,
        grid=(NB,),
        in_specs=[
            pl.BlockSpec((NC, BM, D), lambda i: (0, i, 0)),
            pl.BlockSpec((BM, D), lambda i: (i, 0)),
            pl.BlockSpec((NC, BM, D), lambda i: (0, i, 0)),
            pl.BlockSpec((1, D), lambda i: (0, 0)),
        ],
        out_specs=pl.BlockSpec((8, D), lambda i: (0, 0)),
        out_shape=jax.ShapeDtypeStruct((8, D), jnp.float32),
    )(p, g1, degp, b1)


def _apply_body(p_ref, g1_ref, degp_ref, b1_ref, gm_ref, bt_ref, w2_ref, st_ref, o_ref):
    out1 = _out1_block(p_ref, g1_ref, degp_ref, b1_ref)
    mean = st_ref[0:1, :] * (1.0 / N)
    var = st_ref[1:2, :] * (1.0 / N) - mean * mean
    h = (out1 - mean) * lax.rsqrt(var + 1e-5) * gm_ref[...] + bt_ref[...]
    h = jnp.maximum(h, 0.0)
    o_ref[...] = jnp.dot(h, w2_ref[...], preferred_element_type=jnp.float32) * _dis_block(degp_ref)


def _apply(p, g1, degp, b1, gamma, beta, w2, st):
    return pl.pallas_call(
        _apply_body,
        grid=(NB,),
        in_specs=[
            pl.BlockSpec((NC, BM, D), lambda i: (0, i, 0)),
            pl.BlockSpec((BM, D), lambda i: (i, 0)),
            pl.BlockSpec((NC, BM, D), lambda i: (0, i, 0)),
            pl.BlockSpec((1, D), lambda i: (0, 0)),
            pl.BlockSpec((1, D), lambda i: (0, 0)),
            pl.BlockSpec((1, D), lambda i: (0, 0)),
            pl.BlockSpec((D, D), lambda i: (0, 0)),
            pl.BlockSpec((8, D), lambda i: (0, 0)),
        ],
        out_specs=pl.BlockSpec((BM, D), lambda i: (i, 0)),
        out_shape=jax.ShapeDtypeStruct((N, D), jnp.float32),
    )(p, g1, degp, b1, gamma, beta, w2, st)


def _final_body(q_ref, g2_ref, degp_ref, b2_ref, wt_ref, lb_ref, o_ref):
    out2 = _out1_block(q_ref, g2_ref, degp_ref, b2_ref)
    z = jnp.dot(out2, wt_ref[...], preferred_element_type=jnp.float32) + lb_ref[...]
    m = jnp.max(z, axis=1, keepdims=True)
    zs = z - m
    lse = jnp.log(jnp.sum(jnp.exp(zs), axis=1, keepdims=True))
    o_ref[...] = zs - lse


def _final(q, g2, degp, b2, linWT, linb):
    return pl.pallas_call(
        _final_body,
        grid=(NB,),
        in_specs=[
            pl.BlockSpec((NC, BM, D), lambda i: (0, i, 0)),
            pl.BlockSpec((BM, D), lambda i: (i, 0)),
            pl.BlockSpec((NC, BM, D), lambda i: (0, i, 0)),
            pl.BlockSpec((1, D), lambda i: (0, 0)),
            pl.BlockSpec((D, D), lambda i: (0, 0)),
            pl.BlockSpec((1, D), lambda i: (0, 0)),
        ],
        out_specs=pl.BlockSpec((BM, D), lambda i: (i, 0)),
        out_shape=jax.ShapeDtypeStruct((N, D), jnp.float32),
    )(q, g2, degp, b2, linWT, linb)


def kernel(x, edge_index, W1, b1, gamma, beta, W2, b2, linW, linb):
    src = edge_index[0]
    dst = edge_index[1]
    pad = NW * EPW_PAD - E
    srcp = jnp.concatenate([src, jnp.zeros((pad,), src.dtype)]).reshape(NW, NCHUNK, CHUNK)
    dstp = jnp.concatenate([dst, jnp.full((pad,), N, dst.dtype)]).reshape(NW, NCHUNK, CHUNK)
    padq = TOTCH * CHUNK - E
    srcq = jnp.concatenate([src, jnp.zeros((padq,), src.dtype)]).reshape(TOTCH, CHUNK)
    dstq = jnp.concatenate([dst, jnp.full((padq,), N, dst.dtype)]).reshape(TOTCH, CHUNK)

    zeros = jnp.zeros((N_PAD, D), jnp.float32)
    ones = jnp.ones((CHUNK, D), jnp.float32)

    degp = _sc_degree(dstp, ones, zeros)
    h1 = _mm(x, W1)
    g1 = _scale(h1, degp)
    p = _sc_aggregate(g1, srcq, dstq, zeros)
    b1r = b1.reshape(1, D)
    st = _stats(p, g1, degp, b1r)
    g2 = _apply(p, g1, degp, b1r, gamma.reshape(1, D), beta.reshape(1, D), W2, st)
    q = _sc_aggregate(g2, srcq, dstq, zeros)
    return _final(q, g2, degp, b2.reshape(1, D), linW.T, linb.reshape(1, D))


# speedup vs baseline: 1.6571x; 1.3366x over previous
import functools

import jax
import jax.numpy as jnp
from jax import lax
from jax.experimental import pallas as pl
from jax.experimental.pallas import tpu as pltpu
from jax.experimental.pallas import tpu_sc as plsc

N = 10000
E = 320000
D = 128

NC = 2
NS = 16
NW = NC * NS

CHUNK = 128
EPW = E // NW
NCHUNK = -(-EPW // CHUNK)
EPW_PAD = NCHUNK * CHUNK
RPT = 8 * -(-(N + 1) // (8 * NS))
N_PAD = RPT * NS

BM = 1000
NB = N // BM

@functools.cache
def _mesh():
    return plsc.VectorSubcoreMesh(core_axis_name="c", subcore_axis_name="s",
                                  num_cores=NC, num_subcores=NS)


def _sc_degree(dstp, ones, zeros):
    @functools.partial(
        pl.kernel,
        out_type=jax.ShapeDtypeStruct((NC, N_PAD, D), jnp.float32),
        mesh=_mesh(),
        scratch_types=[
            pltpu.VMEM((NCHUNK, CHUNK), jnp.int32),
            pltpu.VMEM((CHUNK, D), jnp.float32),
            pltpu.VMEM_SHARED((N_PAD, D), jnp.float32),
        ],
    )
    def k(dst_hbm, ones_hbm, z_hbm, out_hbm, dst_v, ones_v, acc_sh):
        c = lax.axis_index("c")
        s = lax.axis_index("s")
        wid = s * NC + c
        pltpu.sync_copy(dst_hbm.at[wid], dst_v)
        pltpu.sync_copy(ones_hbm, ones_v)
        rows = pl.ds(s * RPT, RPT)
        pltpu.sync_copy(z_hbm.at[rows], acc_sh.at[rows])
        plsc.subcore_barrier()

        @pl.loop(0, NCHUNK)
        def _(j):
            pltpu.sync_copy(ones_v, acc_sh.at[dst_v.at[j]], add=True)

        plsc.subcore_barrier()
        pltpu.sync_copy(acc_sh.at[rows], out_hbm.at[c, rows])

    return k(dstp, ones, zeros)


def _sc_aggregate(g, srcp, dstp, zeros):
    @functools.partial(
        pl.kernel,
        out_type=jax.ShapeDtypeStruct((NC, N_PAD, D), jnp.float32),
        mesh=_mesh(),
        scratch_types=[
            pltpu.VMEM((NCHUNK, CHUNK), jnp.int32),
            pltpu.VMEM((NCHUNK, CHUNK), jnp.int32),
            pltpu.VMEM((CHUNK, D), jnp.float32),
            pltpu.VMEM_SHARED((N_PAD, D), jnp.float32),
        ],
    )
    def k(g_hbm, src_hbm, dst_hbm, z_hbm, out_hbm, src_v, dst_v, rows_v, acc_sh):
        c = lax.axis_index("c")
        s = lax.axis_index("s")
        wid = s * NC + c
        rows = pl.ds(s * RPT, RPT)
        pltpu.sync_copy(z_hbm.at[rows], acc_sh.at[rows])
        pltpu.sync_copy(src_hbm.at[wid], src_v)
        pltpu.sync_copy(dst_hbm.at[wid], dst_v)
        plsc.subcore_barrier()

        @pl.loop(0, NCHUNK)
        def _(j):
            pltpu.sync_copy(g_hbm.at[src_v.at[j]], rows_v)
            pltpu.sync_copy(rows_v, acc_sh.at[dst_v.at[j]], add=True)

        plsc.subcore_barrier()
        pltpu.sync_copy(acc_sh.at[rows], out_hbm.at[c, rows])

    return k(g, srcp, dstp, zeros)


def _mm_body(x_ref, w_ref, o_ref):
    o_ref[...] = jnp.dot(x_ref[...], w_ref[...], preferred_element_type=jnp.float32)


def _mm(x, w):
    return pl.pallas_call(
        _mm_body,
        grid=(NB,),
        in_specs=[
            pl.BlockSpec((BM, D), lambda i: (i, 0)),
            pl.BlockSpec((D, D), lambda i: (0, 0)),
        ],
        out_specs=pl.BlockSpec((BM, D), lambda i: (i, 0)),
        out_shape=jax.ShapeDtypeStruct((N, D), jnp.float32),
    )(x, w)


def _dis_block(degp_ref):
    deg = 1.0 + degp_ref[0, :, 0:1]
    for i in range(1, NC):
        deg = deg + degp_ref[i, :, 0:1]
    return lax.rsqrt(deg)


def _scale_body(h_ref, degp_ref, o_ref):
    o_ref[...] = h_ref[...] * _dis_block(degp_ref)


def _scale(h, degp):
    return pl.pallas_call(
        _scale_body,
        grid=(NB,),
        in_specs=[
            pl.BlockSpec((BM, D), lambda i: (i, 0)),
            pl.BlockSpec((NC, BM, D), lambda i: (0, i, 0)),
        ],
        out_specs=pl.BlockSpec((BM, D), lambda i: (i, 0)),
        out_shape=jax.ShapeDtypeStruct((N, D), jnp.float32),
    )(h, degp)


def _out1_block(p_ref, g1_ref, degp_ref, b1_ref):
    agg = p_ref[0] + g1_ref[...]
    for i in range(1, NC):
        agg = agg + p_ref[i]
    return agg * _dis_block(degp_ref) + b1_ref[...]


def _stats_body(p_ref, g1_ref, degp_ref, b1_ref, o_ref):
    i = pl.program_id(0)
    out1 = _out1_block(p_ref, g1_ref, degp_ref, b1_ref)

    @pl.when(i == 0)
    def _():
        o_ref[...] = jnp.zeros_like(o_ref)

    o_ref[0:1, :] += jnp.sum(out1, axis=0, keepdims=True)
    o_ref[1:2, :] += jnp.sum(out1 * out1, axis=0, keepdims=True)


def _stats(p, g1, degp, b1):
    return pl.pallas_call(
        _stats_body,
        grid=(NB,),
        in_specs=[
            pl.BlockSpec((NC, BM, D), lambda i: (0, i, 0)),
            pl.BlockSpec((BM, D), lambda i: (i, 0)),
            pl.BlockSpec((NC, BM, D), lambda i: (0, i, 0)),
            pl.BlockSpec((1, D), lambda i: (0, 0)),
        ],
        out_specs=pl.BlockSpec((8, D), lambda i: (0, 0)),
        out_shape=jax.ShapeDtypeStruct((8, D), jnp.float32),
    )(p, g1, degp, b1)


def _apply_body(p_ref, g1_ref, degp_ref, b1_ref, gm_ref, bt_ref, w2_ref, st_ref, o_ref):
    out1 = _out1_block(p_ref, g1_ref, degp_ref, b1_ref)
    mean = st_ref[0:1, :] * (1.0 / N)
    var = st_ref[1:2, :] * (1.0 / N) - mean * mean
    h = (out1 - mean) * lax.rsqrt(var + 1e-5) * gm_ref[...] + bt_ref[...]
    h = jnp.maximum(h, 0.0)
    o_ref[...] = jnp.dot(h, w2_ref[...], preferred_element_type=jnp.float32) * _dis_block(degp_ref)


def _apply(p, g1, degp, b1, gamma, beta, w2, st):
    return pl.pallas_call(
        _apply_body,
        grid=(NB,),
        in_specs=[
            pl.BlockSpec((NC, BM, D), lambda i: (0, i, 0)),
            pl.BlockSpec((BM, D), lambda i: (i, 0)),
            pl.BlockSpec((NC, BM, D), lambda i: (0, i, 0)),
            pl.BlockSpec((1, D), lambda i: (0, 0)),
            pl.BlockSpec((1, D), lambda i: (0, 0)),
            pl.BlockSpec((1, D), lambda i: (0, 0)),
            pl.BlockSpec((D, D), lambda i: (0, 0)),
            pl.BlockSpec((8, D), lambda i: (0, 0)),
        ],
        out_specs=pl.BlockSpec((BM, D), lambda i: (i, 0)),
        out_shape=jax.ShapeDtypeStruct((N, D), jnp.float32),
    )(p, g1, degp, b1, gamma, beta, w2, st)


def _final_body(q_ref, g2_ref, degp_ref, b2_ref, wt_ref, lb_ref, o_ref):
    out2 = _out1_block(q_ref, g2_ref, degp_ref, b2_ref)
    z = jnp.dot(out2, wt_ref[...], preferred_element_type=jnp.float32) + lb_ref[...]
    m = jnp.max(z, axis=1, keepdims=True)
    zs = z - m
    lse = jnp.log(jnp.sum(jnp.exp(zs), axis=1, keepdims=True))
    o_ref[...] = zs - lse


def _final(q, g2, degp, b2, linWT, linb):
    return pl.pallas_call(
        _final_body,
        grid=(NB,),
        in_specs=[
            pl.BlockSpec((NC, BM, D), lambda i: (0, i, 0)),
            pl.BlockSpec((BM, D), lambda i: (i, 0)),
            pl.BlockSpec((NC, BM, D), lambda i: (0, i, 0)),
            pl.BlockSpec((1, D), lambda i: (0, 0)),
            pl.BlockSpec((D, D), lambda i: (0, 0)),
            pl.BlockSpec((1, D), lambda i: (0, 0)),
        ],
        out_specs=pl.BlockSpec((BM, D), lambda i: (i, 0)),
        out_shape=jax.ShapeDtypeStruct((N, D), jnp.float32),
    )(q, g2, degp, b2, linWT, linb)


def kernel(x, edge_index, W1, b1, gamma, beta, W2, b2, linW, linb):
    src = edge_index[0]
    dst = edge_index[1]
    pad = NW * EPW_PAD - E
    srcp = jnp.concatenate([src, jnp.zeros((pad,), src.dtype)]).reshape(NW, NCHUNK, CHUNK)
    dstp = jnp.concatenate([dst, jnp.full((pad,), N, dst.dtype)]).reshape(NW, NCHUNK, CHUNK)

    zeros = jnp.zeros((N_PAD, D), jnp.float32)
    ones = jnp.ones((CHUNK, D), jnp.float32)

    degp = _sc_degree(dstp, ones, zeros)
    h1 = _mm(x, W1)
    g1 = _scale(h1, degp)
    p = _sc_aggregate(g1, srcp, dstp, zeros)
    b1r = b1.reshape(1, D)
    st = _stats(p, g1, degp, b1r)
    g2 = _apply(p, g1, degp, b1r, gamma.reshape(1, D), beta.reshape(1, D), W2, st)
    q = _sc_aggregate(g2, srcp, dstp, zeros)
    return _final(q, g2, degp, b2.reshape(1, D), linW.T, linb.reshape(1, D))
